# Initial kernel scaffold; baseline (speedup 1.0000x reference)
import jax
import jax.numpy as jnp
from jax.experimental import pallas as pl


def kernel(feats, edge_index, W, attn_l, attn_r, bias):
    raise NotImplementedError("write your pallas kernel here")


# speedup vs baseline: 15.0142x; 15.0142x over previous
import jax
import jax.numpy as jnp
from jax import lax
from jax.experimental import pallas as pl
from jax.experimental.pallas import tpu as pltpu
from jax.experimental.pallas import tpu_sc as plsc

N = 10000
E = 320000
D = 128
NC, NS, L = 2, 16, 16
NW = NC * NS
C = 128
EC = 2560
EPAD = EC * C
CA = EC // NS
CB = EC // NW
NP = 10112
RP = NP // NS
NEG = 0.2


def _matmul_body(f_ref, w_ref, a2_ref, h_ref, elr_ref):
    h = jnp.dot(f_ref[...], w_ref[...], preferred_element_type=jnp.float32)
    h_ref[...] = h
    a2 = a2_ref[...]
    el = jnp.sum(h * a2[0:1, :], axis=1)
    er = jnp.sum(h * a2[1:2, :], axis=1)
    elr_ref[...] = jnp.stack([el, er])


def _final_body(p_ref, b_ref, o_ref):
    o_ref[...] = p_ref[0] + p_ref[1] + b_ref[...]


_SEGS = ((0, 128), (128, 128), (256, 128), (384, 128), (512, 120))


CBK = 16
BA = CA // CBK
BB = CB // CBK


def _sc_body(src_hbm, dst_hbm, el_hbm, er_hbm, h_hbm,
             out_hbm, w_hbm,
             srcv, dstv, wv, el_tab, er_tab, w_buf, rows, out_sh, s_sh,
             sem):
    cid = lax.axis_index("c")
    sid = lax.axis_index("s")
    wid = sid * NC + cid
    base = sid * RP

    z16 = jnp.zeros((L,), jnp.float32)

    def zrow(k, carry):
        for j in range(D // L):
            rows[k, pl.ds(j * L, L)] = z16
        return carry

    lax.fori_loop(0, C, zrow, 0)
    for g in range(C // L):
        w_buf[pl.ds(g * L, L)] = z16
    for off, ln in _SEGS:
        pltpu.sync_copy(rows.at[pl.ds(0, ln)],
                        out_sh.at[pl.ds(base + off, ln)])
        pltpu.sync_copy(w_buf.at[pl.ds(0, ln)],
                        s_sh.at[pl.ds(base + off, ln)])
    pltpu.sync_copy(el_hbm, el_tab)
    pltpu.sync_copy(er_hbm, er_tab)
    plsc.subcore_barrier()

    def phase_a_blk(b, carry):
        row0 = pl.multiple_of(sid * CA + b * CBK, CBK)
        pltpu.sync_copy(src_hbm.at[pl.ds(row0, CBK)], srcv)
        pltpu.sync_copy(dst_hbm.at[pl.ds(row0, CBK)], dstv)

        def chunk(i, icarry):
            for g in range(C // L):
                s16 = srcv[i, pl.ds(g * L, L)]
                d16 = dstv[i, pl.ds(g * L, L)]
                x = (plsc.load_gather(el_tab, [s16])
                     + plsc.load_gather(er_tab, [d16]))
                e = jnp.where(x >= 0.0, x, NEG * x)
                wv[i, pl.ds(g * L, L)] = jnp.exp(e)
            pltpu.sync_copy(wv.at[i], s_sh.at[dstv.at[i]], add=True)
            return icarry

        lax.fori_loop(0, CBK, chunk, 0)
        pltpu.sync_copy(wv, w_hbm.at[pl.ds(row0, CBK)])
        return carry

    lax.fori_loop(0, BA, phase_a_blk, 0)
    plsc.subcore_barrier()
    pltpu.sync_copy(s_sh, el_tab)

    def phase_b_blk(b, carry):
        row0 = pl.multiple_of(wid * CB + b * CBK, CBK)
        pltpu.sync_copy(src_hbm.at[pl.ds(row0, CBK)], srcv)
        pltpu.sync_copy(dst_hbm.at[pl.ds(row0, CBK)], dstv)
        pltpu.sync_copy(w_hbm.at[pl.ds(row0, CBK)], wv)

        def chunk(i, icarry):
            pltpu.async_copy(h_hbm.at[srcv.at[i]], rows, sem).wait()
            for g in range(C // L):
                d16 = dstv[i, pl.ds(g * L, L)]
                sg = plsc.load_gather(el_tab, [d16])
                w16 = wv[i, pl.ds(g * L, L)]
                w_buf[pl.ds(g * L, L)] = w16 / (sg + 1e-9)

            def scale(k, kcarry):
                a = plsc.load_gather(w_buf, [jnp.broadcast_to(k, (L,))])
                for j in range(D // L):
                    rows[k, pl.ds(j * L, L)] = rows[k, pl.ds(j * L, L)] * a
                return kcarry

            lax.fori_loop(0, C, scale, 0)
            pltpu.sync_copy(rows, out_sh.at[dstv.at[i]], add=True)
            return icarry

        lax.fori_loop(0, CBK, chunk, 0)
        return carry

    lax.fori_loop(0, BB, phase_b_blk, 0)
    plsc.subcore_barrier()
    for off, ln in _SEGS:
        pltpu.sync_copy(out_sh.at[pl.ds(base + off, ln)],
                        rows.at[pl.ds(0, ln)])
        pltpu.sync_copy(rows.at[pl.ds(0, ln)],
                        out_hbm.at[cid, pl.ds(base + off, ln)])


def kernel(feats, edge_index, W, attn_l, attn_r, bias):
    src = edge_index[0]
    dst = edge_index[1]
    pad = EPAD - E
    src_p = jnp.concatenate([src, jnp.zeros((pad,), jnp.int32)]).reshape(EC, C)
    dst_p = jnp.concatenate([dst, jnp.full((pad,), N, jnp.int32)]).reshape(EC, C)
    feats_p = jnp.pad(feats, ((0, NP - N), (0, 0)))
    a2 = jnp.stack([attn_l, attn_r])

    h_p, elr = pl.pallas_call(
        _matmul_body,
        grid=(NP // 128,),
        in_specs=[
            pl.BlockSpec((128, D), lambda i: (i, 0)),
            pl.BlockSpec((D, D), lambda i: (0, 0)),
            pl.BlockSpec((2, D), lambda i: (0, 0)),
        ],
        out_specs=[
            pl.BlockSpec((128, D), lambda i: (i, 0)),
            pl.BlockSpec((2, 128), lambda i: (0, i)),
        ],
        out_shape=[
            jax.ShapeDtypeStruct((NP, D), jnp.float32),
            jax.ShapeDtypeStruct((2, NP), jnp.float32),
        ],
    )(feats_p, W, a2)

    sc = pl.kernel(
        _sc_body,
        out_type=[
            jax.ShapeDtypeStruct((NC, NP, D), jnp.float32),
            jax.ShapeDtypeStruct((EC, C), jnp.float32),
        ],
        mesh=plsc.VectorSubcoreMesh(core_axis_name="c", subcore_axis_name="s",
                                    num_cores=NC, num_subcores=NS),
        compiler_params=pltpu.CompilerParams(needs_layout_passes=False),
        scratch_types=[
            pltpu.VMEM((CBK, C), jnp.int32),
            pltpu.VMEM((CBK, C), jnp.int32),
            pltpu.VMEM((CBK, C), jnp.float32),
            pltpu.VMEM((NP,), jnp.float32),
            pltpu.VMEM((NP,), jnp.float32),
            pltpu.VMEM((C,), jnp.float32),
            pltpu.VMEM((C, D), jnp.float32),
            pltpu.VMEM_SHARED((NP, D), jnp.float32),
            pltpu.VMEM_SHARED((NP,), jnp.float32),
            pltpu.SemaphoreType.DMA,
        ],
    )
    partials, _w = sc(src_p, dst_p, elr[0], elr[1], h_p)

    out = pl.pallas_call(
        _final_body,
        grid=(NP // 128,),
        in_specs=[
            pl.BlockSpec((2, 128, D), lambda i: (0, i, 0)),
            pl.BlockSpec((1, D), lambda i: (0, 0)),
        ],
        out_specs=pl.BlockSpec((128, D), lambda i: (i, 0)),
        out_shape=jax.ShapeDtypeStruct((NP, D), jnp.float32),
    )(partials, bias.reshape(1, D))
    return out[:N].reshape(N, 1, D)
